# Initial kernel scaffold; baseline (speedup 1.0000x reference)
import jax
import jax.numpy as jnp
from jax.experimental import pallas as pl


def kernel(x, edge_index, W1, b1, g1, bt1, W2, b2, g2, bt2, W3, b3):
    raise NotImplementedError("write your pallas kernel here")


# speedup vs baseline: 13.0281x; 13.0281x over previous
import functools

import jax
import jax.numpy as jnp
from jax import lax
from jax.experimental import pallas as pl
from jax.experimental.pallas import tpu as pltpu
from jax.experimental.pallas import tpu_sc as plsc

N = 10000
E = 320000
IN_DIM = 128
HID = 64
OUT_DIM = 3

N_PAD = 10112
CH = 128
TILES = 32
PER_TILE = 10112
E_PAD = PER_TILE * TILES
ROWS_PER_TILE = N_PAD // 16


def _make_scatter(D):
    mesh = plsc.VectorSubcoreMesh(core_axis_name="c", subcore_axis_name="s")

    def body(src_hbm, dst_hbm, table_hbm, zeros_hbm, out_hbm,
             acc_sh, src_v, dst_v, rows_v, sem):
        c = lax.axis_index("c")
        s = lax.axis_index("s")
        roff = s * ROWS_PER_TILE
        pltpu.sync_copy(zeros_hbm.at[pl.ds(roff, ROWS_PER_TILE)],
                        acc_sh.at[pl.ds(roff, ROWS_PER_TILE)])
        plsc.subcore_barrier()

        tid = c * 16 + s
        base = tid * PER_TILE

        def chunk(i, carry):
            off = base + i * CH
            pltpu.sync_copy(src_hbm.at[pl.ds(off, CH)], src_v)
            pltpu.sync_copy(dst_hbm.at[pl.ds(off, CH)], dst_v)
            pltpu.async_copy(table_hbm.at[src_v], rows_v, sem).wait()
            pltpu.sync_copy(rows_v, acc_sh.at[dst_v], add=True)
            return carry

        lax.fori_loop(0, PER_TILE // CH, chunk, 0)
        plsc.subcore_barrier()
        pltpu.sync_copy(acc_sh.at[pl.ds(roff, ROWS_PER_TILE)],
                        out_hbm.at[c, pl.ds(roff, ROWS_PER_TILE)])

    return pl.kernel(
        body,
        out_type=jax.ShapeDtypeStruct((2, N_PAD, D), jnp.float32),
        mesh=mesh,
        scratch_types=[
            pltpu.VMEM_SHARED((N_PAD, D), jnp.float32),
            pltpu.VMEM((CH,), jnp.int32),
            pltpu.VMEM((CH,), jnp.int32),
            pltpu.VMEM((CH, D), jnp.float32),
            pltpu.SemaphoreType.DMA,
        ],
        compiler_params=pltpu.CompilerParams(use_tc_tiling_on_sc=False),
    )


_scat8 = _make_scatter(8)
_scat64 = _make_scatter(HID)


def _tc1_body(parts_ref, x_ref, w1_ref, dinv_ref, u1_ref):
    deg = parts_ref[0, :N, 0:1] + parts_ref[1, :N, 0:1] + 1.0
    dinv = lax.rsqrt(deg)
    dinv_ref[...] = dinv
    xw = jnp.dot(x_ref[...], w1_ref[...], preferred_element_type=jnp.float32)
    u1_ref[...] = dinv * xw


_tc1 = pl.pallas_call(
    _tc1_body,
    out_shape=(jax.ShapeDtypeStruct((N, 1), jnp.float32),
               jax.ShapeDtypeStruct((N, HID), jnp.float32)),
)


def _mid_body(parts_ref, u_ref, dinv_ref, b_ref, g_ref, bt_ref, w_ref, out_ref):
    dinv = dinv_ref[...]
    z = dinv * (parts_ref[0, :N, :] + parts_ref[1, :N, :] + u_ref[...]) + b_ref[...]
    mu = jnp.mean(z, axis=0, keepdims=True)
    var = jnp.mean((z - mu) ** 2, axis=0, keepdims=True)
    h = g_ref[...] * (z - mu) * lax.rsqrt(var + 1e-5) + bt_ref[...]
    h = jnp.maximum(h, 0.0)
    out_ref[...] = dinv * jnp.dot(h, w_ref[...], preferred_element_type=jnp.float32)


def _make_mid(d_out):
    return pl.pallas_call(
        _mid_body,
        out_shape=jax.ShapeDtypeStruct((N, d_out), jnp.float32),
    )


_mid64 = _make_mid(HID)
_mid8 = _make_mid(8)


def _final_body(parts_ref, u3_ref, dinv_ref, b3_ref, out_ref):
    z = dinv_ref[...] * (parts_ref[0, :N, :] + parts_ref[1, :N, :] + u3_ref[...]) + b3_ref[...]
    z3 = z[:, :OUT_DIM]
    m = jnp.max(z3, axis=1, keepdims=True)
    e = jnp.exp(z3 - m)
    lse = jnp.log(jnp.sum(e, axis=1, keepdims=True))
    out_ref[...] = (z3 - m) - lse


_final = pl.pallas_call(
    _final_body,
    out_shape=jax.ShapeDtypeStruct((N, OUT_DIM), jnp.float32),
)


def kernel(x, edge_index, W1, b1, g1, bt1, W2, b2, g2, bt2, W3, b3):
    ei = edge_index.astype(jnp.int32)
    src = jnp.concatenate([ei[0], jnp.zeros((E_PAD - E,), jnp.int32)])
    dst = jnp.concatenate([ei[1], jnp.full((E_PAD - E,), N, jnp.int32)])
    zeros8 = jnp.zeros((N_PAD, 8), jnp.float32)
    zeros64 = jnp.zeros((N_PAD, HID), jnp.float32)
    ones8 = jnp.ones((N, 8), jnp.float32)

    deg_parts = _scat8(src, dst, ones8, zeros8)
    dinv, u1 = _tc1(deg_parts, x, W1)

    s1 = _scat64(src, dst, u1, zeros64)
    u2 = _mid64(s1, u1, dinv, b1.reshape(1, -1), g1.reshape(1, -1),
                bt1.reshape(1, -1), W2)

    s2 = _scat64(src, dst, u2, zeros64)
    w3p = jnp.pad(W3, ((0, 0), (0, 8 - OUT_DIM)))
    u3 = _mid8(s2, u2, dinv, b2.reshape(1, -1), g2.reshape(1, -1),
               bt2.reshape(1, -1), w3p)

    s3 = _scat8(src, dst, u3, zeros8)
    b3p = jnp.pad(b3, (0, 8 - OUT_DIM)).reshape(1, -1)
    return _final(s3, u3, dinv, b3p)


# speedup vs baseline: 14.8283x; 1.1382x over previous
import jax
import jax.numpy as jnp
from jax import lax
from jax.experimental import pallas as pl
from jax.experimental.pallas import tpu as pltpu
from jax.experimental.pallas import tpu_sc as plsc

N = 10000
E = 320000
IN_DIM = 128
HID = 64
OUT_DIM = 3

N_PAD = 10112
CH = 128
CPT = 80
E_PAD = 32 * CPT * CH
KB = 8
NBLK = CPT // KB
ROWS_PER_TILE = N_PAD // 16


def _make_scatter(D):
    mesh = plsc.VectorSubcoreMesh(core_axis_name="c", subcore_axis_name="s")

    def body(src_hbm, dst_hbm, table_hbm, zeros_hbm, out_hbm,
             acc_sh, src_v, dst_v, rows_v, gsem, ssem):
        c = lax.axis_index("c")
        s = lax.axis_index("s")
        roff = s * ROWS_PER_TILE
        pltpu.sync_copy(zeros_hbm.at[pl.ds(roff, ROWS_PER_TILE)],
                        acc_sh.at[pl.ds(roff, ROWS_PER_TILE)])
        tid = c * 16 + s
        crow = tid * CPT
        pltpu.sync_copy(src_hbm.at[pl.ds(crow, CPT)], src_v)
        pltpu.sync_copy(dst_hbm.at[pl.ds(crow, CPT)], dst_v)
        plsc.subcore_barrier()

        def blk(o, carry):
            jb = o * KB
            gd = [pltpu.async_copy(table_hbm.at[src_v.at[jb + b]],
                                   rows_v.at[b], gsem) for b in range(KB)]
            for d in gd:
                d.wait()
            sd = [pltpu.async_copy(rows_v.at[b], acc_sh.at[dst_v.at[jb + b]],
                                   ssem, add=True) for b in range(KB)]
            for d in sd:
                d.wait()
            return carry

        lax.fori_loop(0, NBLK, blk, 0)
        plsc.subcore_barrier()
        pltpu.sync_copy(acc_sh.at[pl.ds(roff, ROWS_PER_TILE)],
                        out_hbm.at[c, pl.ds(roff, ROWS_PER_TILE)])

    return pl.kernel(
        body,
        out_type=jax.ShapeDtypeStruct((2, N_PAD, D), jnp.float32),
        mesh=mesh,
        scratch_types=[
            pltpu.VMEM_SHARED((N_PAD, D), jnp.float32),
            pltpu.VMEM((CPT, CH), jnp.int32),
            pltpu.VMEM((CPT, CH), jnp.int32),
            pltpu.VMEM((KB, CH, D), jnp.float32),
            pltpu.SemaphoreType.DMA,
            pltpu.SemaphoreType.DMA,
        ],
        compiler_params=pltpu.CompilerParams(use_tc_tiling_on_sc=False),
    )


_scat8 = _make_scatter(8)
_scat64 = _make_scatter(HID)


def _deg_body(dst_hbm, ones_hbm, zeros_hbm, out_hbm, acc_sh, dst_v, ones_v, ssem):
    c = lax.axis_index("c")
    s = lax.axis_index("s")
    roff = s * ROWS_PER_TILE
    pltpu.sync_copy(zeros_hbm.at[pl.ds(roff, ROWS_PER_TILE)],
                    acc_sh.at[pl.ds(roff, ROWS_PER_TILE)])
    tid = c * 16 + s
    pltpu.sync_copy(dst_hbm.at[pl.ds(tid * CPT, CPT)], dst_v)
    pltpu.sync_copy(ones_hbm, ones_v)
    plsc.subcore_barrier()

    def blk(o, carry):
        jb = o * KB
        sd = [pltpu.async_copy(ones_v, acc_sh.at[dst_v.at[jb + b]],
                               ssem, add=True) for b in range(KB)]
        for d in sd:
            d.wait()
        return carry

    lax.fori_loop(0, NBLK, blk, 0)
    plsc.subcore_barrier()
    pltpu.sync_copy(acc_sh.at[pl.ds(roff, ROWS_PER_TILE)],
                    out_hbm.at[c, pl.ds(roff, ROWS_PER_TILE)])


_deg_scat = pl.kernel(
    _deg_body,
    out_type=jax.ShapeDtypeStruct((2, N_PAD, 8), jnp.float32),
    mesh=plsc.VectorSubcoreMesh(core_axis_name="c", subcore_axis_name="s"),
    scratch_types=[
        pltpu.VMEM_SHARED((N_PAD, 8), jnp.float32),
        pltpu.VMEM((CPT, CH), jnp.int32),
        pltpu.VMEM((CH, 8), jnp.float32),
        pltpu.SemaphoreType.DMA,
    ],
    compiler_params=pltpu.CompilerParams(use_tc_tiling_on_sc=False),
)


def _tc1_body(parts_ref, x_ref, w1_ref, dinv_ref, u1_ref):
    deg = parts_ref[0, :N, 0:1] + parts_ref[1, :N, 0:1] + 1.0
    dinv = lax.rsqrt(deg)
    dinv_ref[...] = dinv
    xw = jnp.dot(x_ref[...], w1_ref[...], preferred_element_type=jnp.float32)
    u1_ref[...] = dinv * xw


_tc1 = pl.pallas_call(
    _tc1_body,
    out_shape=(jax.ShapeDtypeStruct((N, 1), jnp.float32),
               jax.ShapeDtypeStruct((N, HID), jnp.float32)),
)


def _mid_body(parts_ref, u_ref, dinv_ref, b_ref, g_ref, bt_ref, w_ref, out_ref):
    dinv = dinv_ref[...]
    z = dinv * (parts_ref[0, :N, :] + parts_ref[1, :N, :] + u_ref[...]) + b_ref[...]
    mu = jnp.mean(z, axis=0, keepdims=True)
    var = jnp.mean((z - mu) ** 2, axis=0, keepdims=True)
    h = g_ref[...] * (z - mu) * lax.rsqrt(var + 1e-5) + bt_ref[...]
    h = jnp.maximum(h, 0.0)
    out_ref[...] = dinv * jnp.dot(h, w_ref[...], preferred_element_type=jnp.float32)


def _make_mid(d_out):
    return pl.pallas_call(
        _mid_body,
        out_shape=jax.ShapeDtypeStruct((N, d_out), jnp.float32),
    )


_mid64 = _make_mid(HID)
_mid8 = _make_mid(8)


def _final_body(parts_ref, u3_ref, dinv_ref, b3_ref, out_ref):
    z = dinv_ref[...] * (parts_ref[0, :N, :] + parts_ref[1, :N, :] + u3_ref[...]) + b3_ref[...]
    z3 = z[:, :OUT_DIM]
    m = jnp.max(z3, axis=1, keepdims=True)
    e = jnp.exp(z3 - m)
    lse = jnp.log(jnp.sum(e, axis=1, keepdims=True))
    out_ref[...] = (z3 - m) - lse


_final = pl.pallas_call(
    _final_body,
    out_shape=jax.ShapeDtypeStruct((N, OUT_DIM), jnp.float32),
)


def kernel(x, edge_index, W1, b1, g1, bt1, W2, b2, g2, bt2, W3, b3):
    ei = edge_index.astype(jnp.int32)
    src = jnp.concatenate([ei[0], jnp.zeros((E_PAD - E,), jnp.int32)])
    dst = jnp.concatenate([ei[1], jnp.full((E_PAD - E,), N, jnp.int32)])
    src2 = src.reshape(E_PAD // CH, CH)
    dst2 = dst.reshape(E_PAD // CH, CH)
    zeros8 = jnp.zeros((N_PAD, 8), jnp.float32)
    zeros64 = jnp.zeros((N_PAD, HID), jnp.float32)
    ones8 = jnp.ones((CH, 8), jnp.float32)

    deg_parts = _deg_scat(dst2, ones8, zeros8)
    dinv, u1 = _tc1(deg_parts, x, W1)

    s1 = _scat64(src2, dst2, u1, zeros64)
    u2 = _mid64(s1, u1, dinv, b1.reshape(1, -1), g1.reshape(1, -1),
                bt1.reshape(1, -1), W2)

    s2 = _scat64(src2, dst2, u2, zeros64)
    w3p = jnp.pad(W3, ((0, 0), (0, 8 - OUT_DIM)))
    u3 = _mid8(s2, u2, dinv, b2.reshape(1, -1), g2.reshape(1, -1),
               bt2.reshape(1, -1), w3p)

    s3 = _scat8(src2, dst2, u3, zeros8)
    b3p = jnp.pad(b3, (0, 8 - OUT_DIM)).reshape(1, -1)
    return _final(s3, u3, dinv, b3p)


# speedup vs baseline: 32.2417x; 2.1743x over previous
import jax
import jax.numpy as jnp
from jax import lax
from jax.experimental import pallas as pl
from jax.experimental.pallas import tpu as pltpu
from jax.experimental.pallas import tpu_sc as plsc

N = 10000
E = 320000
IN_DIM = 128
HID = 64
OUT_DIM = 3

N_PAD = 10112
CH = 128
CPT = 80
E_PAD = 32 * CPT * CH
KB = 8
NBLK = CPT // KB
KBG = 4
SB = 40
NBLKG = CPT // KBG
ROWS_PER_TILE = N_PAD // 16


def _make_scatter(D):
    mesh = plsc.VectorSubcoreMesh(core_axis_name="c", subcore_axis_name="s")

    def body(src_hbm, dst_hbm, table_hbm, zeros_hbm, out_hbm,
             big_sh, src_v, dst_v, rows_v, gsem, ssem):
        c = lax.axis_index("c")
        s = lax.axis_index("s")
        roff = s * ROWS_PER_TILE
        pltpu.sync_copy(zeros_hbm.at[pl.ds(roff, ROWS_PER_TILE)],
                        big_sh.at[pl.ds(roff, ROWS_PER_TILE)])
        toff = s * (N // 16)
        pltpu.sync_copy(table_hbm.at[pl.ds(toff, N // 16)],
                        big_sh.at[pl.ds(N_PAD + toff, N // 16)])
        tid = c * 16 + s
        crow = tid * CPT
        plsc.subcore_barrier()

        def blk(o, carry):
            @pl.when(lax.rem(o, SB // KBG) == 0)
            def _():
                srow = crow + (o // (SB // KBG)) * SB
                pltpu.sync_copy(src_hbm.at[pl.ds(srow, SB)], src_v)
                pltpu.sync_copy(dst_hbm.at[pl.ds(srow, SB)], dst_v)

            jb = lax.rem(o, SB // KBG) * KBG
            gd = [pltpu.async_copy(big_sh.at[src_v.at[jb + b]],
                                   rows_v.at[b], gsem) for b in range(KBG)]
            for d in gd:
                d.wait()
            sd = [pltpu.async_copy(rows_v.at[b], big_sh.at[dst_v.at[jb + b]],
                                   ssem, add=True) for b in range(KBG)]
            for d in sd:
                d.wait()
            return carry

        lax.fori_loop(0, NBLKG, blk, 0)
        plsc.subcore_barrier()
        pltpu.sync_copy(big_sh.at[pl.ds(roff, ROWS_PER_TILE)],
                        out_hbm.at[c, pl.ds(roff, ROWS_PER_TILE)])

    return pl.kernel(
        body,
        out_type=jax.ShapeDtypeStruct((2, N_PAD, D), jnp.float32),
        mesh=mesh,
        scratch_types=[
            pltpu.VMEM_SHARED((N_PAD + N, D), jnp.float32),
            pltpu.VMEM((SB, CH), jnp.int32),
            pltpu.VMEM((SB, CH), jnp.int32),
            pltpu.VMEM((KBG, CH, D), jnp.float32),
            pltpu.SemaphoreType.DMA,
            pltpu.SemaphoreType.DMA,
        ],
        compiler_params=pltpu.CompilerParams(use_tc_tiling_on_sc=False),
    )


_scat8 = _make_scatter(8)
_scat64 = _make_scatter(HID)


def _deg_body(dst_hbm, ones_hbm, zeros_hbm, out_hbm, acc_sh, dst_v, ones_v, ssem):
    c = lax.axis_index("c")
    s = lax.axis_index("s")
    roff = s * ROWS_PER_TILE
    pltpu.sync_copy(zeros_hbm.at[pl.ds(roff, ROWS_PER_TILE)],
                    acc_sh.at[pl.ds(roff, ROWS_PER_TILE)])
    tid = c * 16 + s
    pltpu.sync_copy(dst_hbm.at[pl.ds(tid * CPT, CPT)], dst_v)
    pltpu.sync_copy(ones_hbm, ones_v)
    plsc.subcore_barrier()

    def blk(o, carry):
        jb = o * KB
        sd = [pltpu.async_copy(ones_v, acc_sh.at[dst_v.at[jb + b]],
                               ssem, add=True) for b in range(KB)]
        for d in sd:
            d.wait()
        return carry

    lax.fori_loop(0, NBLK, blk, 0)
    plsc.subcore_barrier()
    pltpu.sync_copy(acc_sh.at[pl.ds(roff, ROWS_PER_TILE)],
                    out_hbm.at[c, pl.ds(roff, ROWS_PER_TILE)])


_deg_scat = pl.kernel(
    _deg_body,
    out_type=jax.ShapeDtypeStruct((2, N_PAD, 8), jnp.float32),
    mesh=plsc.VectorSubcoreMesh(core_axis_name="c", subcore_axis_name="s"),
    scratch_types=[
        pltpu.VMEM_SHARED((N_PAD, 8), jnp.float32),
        pltpu.VMEM((CPT, CH), jnp.int32),
        pltpu.VMEM((CH, 8), jnp.float32),
        pltpu.SemaphoreType.DMA,
    ],
    compiler_params=pltpu.CompilerParams(use_tc_tiling_on_sc=False),
)


def _tc1_body(parts_ref, x_ref, w1_ref, dinv_ref, u1_ref):
    deg = parts_ref[0, :N, 0:1] + parts_ref[1, :N, 0:1] + 1.0
    dinv = lax.rsqrt(deg)
    dinv_ref[...] = dinv
    xw = jnp.dot(x_ref[...], w1_ref[...], preferred_element_type=jnp.float32)
    u1_ref[...] = dinv * xw


_tc1 = pl.pallas_call(
    _tc1_body,
    out_shape=(jax.ShapeDtypeStruct((N, 1), jnp.float32),
               jax.ShapeDtypeStruct((N, HID), jnp.float32)),
)


def _mid_body(parts_ref, u_ref, dinv_ref, b_ref, g_ref, bt_ref, w_ref, out_ref):
    dinv = dinv_ref[...]
    z = dinv * (parts_ref[0, :N, :] + parts_ref[1, :N, :] + u_ref[...]) + b_ref[...]
    mu = jnp.mean(z, axis=0, keepdims=True)
    var = jnp.mean((z - mu) ** 2, axis=0, keepdims=True)
    h = g_ref[...] * (z - mu) * lax.rsqrt(var + 1e-5) + bt_ref[...]
    h = jnp.maximum(h, 0.0)
    out_ref[...] = dinv * jnp.dot(h, w_ref[...], preferred_element_type=jnp.float32)


def _make_mid(d_out):
    return pl.pallas_call(
        _mid_body,
        out_shape=jax.ShapeDtypeStruct((N, d_out), jnp.float32),
    )


_mid64 = _make_mid(HID)
_mid8 = _make_mid(8)


def _final_body(parts_ref, u3_ref, dinv_ref, b3_ref, out_ref):
    z = dinv_ref[...] * (parts_ref[0, :N, :] + parts_ref[1, :N, :] + u3_ref[...]) + b3_ref[...]
    z3 = z[:, :OUT_DIM]
    m = jnp.max(z3, axis=1, keepdims=True)
    e = jnp.exp(z3 - m)
    lse = jnp.log(jnp.sum(e, axis=1, keepdims=True))
    out_ref[...] = (z3 - m) - lse


_final = pl.pallas_call(
    _final_body,
    out_shape=jax.ShapeDtypeStruct((N, OUT_DIM), jnp.float32),
)


def kernel(x, edge_index, W1, b1, g1, bt1, W2, b2, g2, bt2, W3, b3):
    ei = edge_index.astype(jnp.int32)
    src = jnp.concatenate([ei[0], jnp.zeros((E_PAD - E,), jnp.int32)])
    dst = jnp.concatenate([ei[1], jnp.full((E_PAD - E,), N, jnp.int32)])
    src2 = src.reshape(E_PAD // CH, CH) + N_PAD
    dst2 = dst.reshape(E_PAD // CH, CH)
    zeros8 = jnp.zeros((N_PAD, 8), jnp.float32)
    zeros64 = jnp.zeros((N_PAD, HID), jnp.float32)
    ones8 = jnp.ones((CH, 8), jnp.float32)

    deg_parts = _deg_scat(dst2, ones8, zeros8)
    dinv, u1 = _tc1(deg_parts, x, W1)

    s1 = _scat64(src2, dst2, u1, zeros64)
    u2 = _mid64(s1, u1, dinv, b1.reshape(1, -1), g1.reshape(1, -1),
                bt1.reshape(1, -1), W2)

    s2 = _scat64(src2, dst2, u2, zeros64)
    w3p = jnp.pad(W3, ((0, 0), (0, 8 - OUT_DIM)))
    u3 = _mid8(s2, u2, dinv, b2.reshape(1, -1), g2.reshape(1, -1),
               bt2.reshape(1, -1), w3p)

    s3 = _scat8(src2, dst2, u3, zeros8)
    b3p = jnp.pad(b3, (0, 8 - OUT_DIM)).reshape(1, -1)
    return _final(s3, u3, dinv, b3p)


# speedup vs baseline: 38.9850x; 1.2091x over previous
import jax
import jax.numpy as jnp
from jax import lax
from jax.experimental import pallas as pl
from jax.experimental.pallas import tpu as pltpu
from jax.experimental.pallas import tpu_sc as plsc

N = 10000
E = 320000
IN_DIM = 128
HID = 64
OUT_DIM = 3

N_PAD = 10112
CH = 128
CPT = 80
E_PAD = 32 * CPT * CH
KB = 8
NBLK = CPT // KB
KBG = 4
SB = 20
ROWS_PER_TILE = N_PAD // 16


def _make_scatter(D):
    mesh = plsc.VectorSubcoreMesh(core_axis_name="c", subcore_axis_name="s")

    def body(src_hbm, dst_hbm, table_hbm, zeros_hbm, out_hbm,
             big_sh, src_v, dst_v, rows_v, gsem, ssem):
        c = lax.axis_index("c")
        s = lax.axis_index("s")
        roff = s * ROWS_PER_TILE
        pltpu.sync_copy(zeros_hbm.at[pl.ds(roff, ROWS_PER_TILE)],
                        big_sh.at[pl.ds(roff, ROWS_PER_TILE)])
        toff = s * (N // 16)
        pltpu.sync_copy(table_hbm.at[pl.ds(toff, N // 16)],
                        big_sh.at[pl.ds(N_PAD + toff, N // 16)])
        tid = c * 16 + s
        crow = tid * CPT
        plsc.subcore_barrier()

        def load_sb(sb):
            srow = crow + sb * SB
            slot = lax.rem(sb, 2)
            pltpu.sync_copy(src_hbm.at[pl.ds(srow, SB)], src_v.at[slot])
            pltpu.sync_copy(dst_hbm.at[pl.ds(srow, SB)], dst_v.at[slot])

        def g_desc(j):
            return pltpu.make_async_copy(
                big_sh.at[src_v.at[lax.rem(j // SB, 2), lax.rem(j, SB)]],
                rows_v.at[lax.rem(j, KBG)], gsem)

        def s_wait(j):
            pltpu.make_async_copy(rows_v.at[lax.rem(j, KBG)],
                                  big_sh.at[dst_v.at[lax.rem(j // SB, 2),
                                                     lax.rem(j, SB)]],
                                  ssem).wait()

        load_sb(0)
        g_desc(0).start()
        g_desc(1).start()

        def it(j, carry):
            @pl.when(j >= 2)
            def _():
                s_wait(j - 2)

            jn = j + 2
            @pl.when(jn < CPT)
            def _():
                @pl.when(lax.rem(jn, SB) == 0)
                def _():
                    load_sb(jn // SB)
                g_desc(jn).start()

            g_desc(j).wait()
            pltpu.async_copy(rows_v.at[lax.rem(j, KBG)],
                             big_sh.at[dst_v.at[lax.rem(j // SB, 2),
                                                lax.rem(j, SB)]],
                             ssem, add=True)
            return carry

        lax.fori_loop(0, CPT, it, 0)
        s_wait(CPT - 2)
        s_wait(CPT - 1)
        plsc.subcore_barrier()
        pltpu.sync_copy(big_sh.at[pl.ds(roff, ROWS_PER_TILE)],
                        out_hbm.at[c, pl.ds(roff, ROWS_PER_TILE)])

    return pl.kernel(
        body,
        out_type=jax.ShapeDtypeStruct((2, N_PAD, D), jnp.float32),
        mesh=mesh,
        scratch_types=[
            pltpu.VMEM_SHARED((N_PAD + N, D), jnp.float32),
            pltpu.VMEM((2, SB, CH), jnp.int32),
            pltpu.VMEM((2, SB, CH), jnp.int32),
            pltpu.VMEM((KBG, CH, D), jnp.float32),
            pltpu.SemaphoreType.DMA,
            pltpu.SemaphoreType.DMA,
        ],
        compiler_params=pltpu.CompilerParams(use_tc_tiling_on_sc=False),
    )


_scat8 = _make_scatter(8)
_scat64 = _make_scatter(HID)


def _deg_body(dst_hbm, ones_hbm, zeros_hbm, out_hbm, acc_sh, dst_v, ones_v, ssem):
    c = lax.axis_index("c")
    s = lax.axis_index("s")
    roff = s * ROWS_PER_TILE
    pltpu.sync_copy(zeros_hbm.at[pl.ds(roff, ROWS_PER_TILE)],
                    acc_sh.at[pl.ds(roff, ROWS_PER_TILE)])
    tid = c * 16 + s
    pltpu.sync_copy(dst_hbm.at[pl.ds(tid * CPT, CPT)], dst_v)
    pltpu.sync_copy(ones_hbm, ones_v)
    plsc.subcore_barrier()

    def blk(o, carry):
        jb = o * KB
        sd = [pltpu.async_copy(ones_v, acc_sh.at[dst_v.at[jb + b]],
                               ssem, add=True) for b in range(KB)]
        for d in sd:
            d.wait()
        return carry

    lax.fori_loop(0, NBLK, blk, 0)
    plsc.subcore_barrier()
    pltpu.sync_copy(acc_sh.at[pl.ds(roff, ROWS_PER_TILE)],
                    out_hbm.at[c, pl.ds(roff, ROWS_PER_TILE)])


_deg_scat = pl.kernel(
    _deg_body,
    out_type=jax.ShapeDtypeStruct((2, N_PAD, 8), jnp.float32),
    mesh=plsc.VectorSubcoreMesh(core_axis_name="c", subcore_axis_name="s"),
    scratch_types=[
        pltpu.VMEM_SHARED((N_PAD, 8), jnp.float32),
        pltpu.VMEM((CPT, CH), jnp.int32),
        pltpu.VMEM((CH, 8), jnp.float32),
        pltpu.SemaphoreType.DMA,
    ],
    compiler_params=pltpu.CompilerParams(use_tc_tiling_on_sc=False),
)


def _tc1_body(parts_ref, x_ref, w1_ref, dinv_ref, u1_ref):
    deg = parts_ref[0, :N, 0:1] + parts_ref[1, :N, 0:1] + 1.0
    dinv = lax.rsqrt(deg)
    dinv_ref[...] = dinv
    xw = jnp.dot(x_ref[...], w1_ref[...], preferred_element_type=jnp.float32)
    u1_ref[...] = dinv * xw


_tc1 = pl.pallas_call(
    _tc1_body,
    out_shape=(jax.ShapeDtypeStruct((N, 1), jnp.float32),
               jax.ShapeDtypeStruct((N, HID), jnp.float32)),
)


def _mid_body(parts_ref, u_ref, dinv_ref, b_ref, g_ref, bt_ref, w_ref, out_ref):
    dinv = dinv_ref[...]
    z = dinv * (parts_ref[0, :N, :] + parts_ref[1, :N, :] + u_ref[...]) + b_ref[...]
    mu = jnp.mean(z, axis=0, keepdims=True)
    var = jnp.mean((z - mu) ** 2, axis=0, keepdims=True)
    h = g_ref[...] * (z - mu) * lax.rsqrt(var + 1e-5) + bt_ref[...]
    h = jnp.maximum(h, 0.0)
    out_ref[...] = dinv * jnp.dot(h, w_ref[...], preferred_element_type=jnp.float32)


def _make_mid(d_out):
    return pl.pallas_call(
        _mid_body,
        out_shape=jax.ShapeDtypeStruct((N, d_out), jnp.float32),
    )


_mid64 = _make_mid(HID)
_mid8 = _make_mid(8)


def _final_body(parts_ref, u3_ref, dinv_ref, b3_ref, out_ref):
    z = dinv_ref[...] * (parts_ref[0, :N, :] + parts_ref[1, :N, :] + u3_ref[...]) + b3_ref[...]
    z3 = z[:, :OUT_DIM]
    m = jnp.max(z3, axis=1, keepdims=True)
    e = jnp.exp(z3 - m)
    lse = jnp.log(jnp.sum(e, axis=1, keepdims=True))
    out_ref[...] = (z3 - m) - lse


_final = pl.pallas_call(
    _final_body,
    out_shape=jax.ShapeDtypeStruct((N, OUT_DIM), jnp.float32),
)


def kernel(x, edge_index, W1, b1, g1, bt1, W2, b2, g2, bt2, W3, b3):
    ei = edge_index.astype(jnp.int32)
    src = jnp.concatenate([ei[0], jnp.zeros((E_PAD - E,), jnp.int32)])
    dst = jnp.concatenate([ei[1], jnp.full((E_PAD - E,), N, jnp.int32)])
    src2 = src.reshape(E_PAD // CH, CH) + N_PAD
    dst2 = dst.reshape(E_PAD // CH, CH)
    zeros8 = jnp.zeros((N_PAD, 8), jnp.float32)
    zeros64 = jnp.zeros((N_PAD, HID), jnp.float32)
    ones8 = jnp.ones((CH, 8), jnp.float32)

    deg_parts = _deg_scat(dst2, ones8, zeros8)
    dinv, u1 = _tc1(deg_parts, x, W1)

    s1 = _scat64(src2, dst2, u1, zeros64)
    u2 = _mid64(s1, u1, dinv, b1.reshape(1, -1), g1.reshape(1, -1),
                bt1.reshape(1, -1), W2)

    s2 = _scat64(src2, dst2, u2, zeros64)
    w3p = jnp.pad(W3, ((0, 0), (0, 8 - OUT_DIM)))
    u3 = _mid8(s2, u2, dinv, b2.reshape(1, -1), g2.reshape(1, -1),
               bt2.reshape(1, -1), w3p)

    s3 = _scat8(src2, dst2, u3, zeros8)
    b3p = jnp.pad(b3, (0, 8 - OUT_DIM)).reshape(1, -1)
    return _final(s3, u3, dinv, b3p)
